# baseline (speedup 1.0000x reference)
import functools

import jax
import jax.numpy as jnp
from jax import lax
from jax.experimental import pallas as pl
from jax.experimental.pallas import tpu as pltpu
from jax.experimental.pallas import tpu_sc as plsc

B, L, E, H = 1024, 50, 64, 256
N = B * L
NF = 4
CAT_W = NF * E
DENSE_W = 150
SMALL = 145
NC, NS = 2, 16
NW = NC * NS
BPW = N // NW
CH = 80
NCH = BPW // CH
TM = 800
NB = N // TM


def _sc_gather_body(*refs):
  idxs = refs[:NF]
  tables = refs[NF:2 * NF]
  outs = refs[2 * NF:3 * NF]
  idx_v, rows_v, sem = refs[3 * NF:]
  wid = lax.axis_index("s") * NC + lax.axis_index("c")
  base = pl.multiple_of(wid * BPW, 8)
  for f in range(NF):
    tab = tables[f]
    pltpu.sync_copy(idxs[f].at[pl.ds(base, BPW)], idx_v)

    def _fire(c, carry, tab=tab):
      off = pl.multiple_of(c * CH, 8)
      pltpu.make_async_copy(
          tab.at[idx_v.at[pl.ds(off, CH)]],
          rows_v.at[pl.ds(off, CH), :],
          sem).start()
      return carry

    def _drain(c, carry, tab=tab):
      off = pl.multiple_of(c * CH, 8)
      pltpu.make_async_copy(
          tab.at[idx_v.at[pl.ds(off, CH)]],
          rows_v.at[pl.ds(off, CH), :],
          sem).wait()
      return carry

    lax.fori_loop(0, NCH, _fire, 0, unroll=False)
    lax.fori_loop(0, NCH, _drain, 0, unroll=False)
    pltpu.sync_copy(rows_v, outs[f].at[pl.ds(base, BPW)])


def _sc_gather(idx_list, tables):
  mesh = plsc.VectorSubcoreMesh(core_axis_name="c", subcore_axis_name="s")
  fn = pl.kernel(
      _sc_gather_body,
      mesh=mesh,
      out_type=[jax.ShapeDtypeStruct((N, E), jnp.float32)] * NF,
      scratch_types=[
          pltpu.VMEM((BPW,), jnp.int32),
          pltpu.VMEM((BPW, E), jnp.float32),
          pltpu.SemaphoreType.DMA,
      ],
      compiler_params=pltpu.CompilerParams(use_tc_tiling_on_sc=False),
  )
  return fn(*idx_list, *tables)


def _tc_partial_body(sidx_ref, dense_ref, w2_ref, p_ref, bp_ref, out_ref):
  acc = jnp.dot(dense_ref[...], w2_ref[...],
                preferred_element_type=jnp.float32)
  pos_iota = lax.broadcasted_iota(jnp.int32, (1, SMALL), 1)
  sall = sidx_ref[...]
  oh = jnp.zeros((TM, SMALL), dtype=jnp.float32)
  for f in range(9):
    idx_f = sall[0, :, f:f + 1]
    oh = oh + (idx_f == pos_iota).astype(jnp.float32)
  acc = acc + jnp.dot(oh, p_ref[...], preferred_element_type=jnp.float32)
  out_ref[...] = acc + bp_ref[...]


def _tc_partial(sidx, dense, w2, p, bp):
  return pl.pallas_call(
      _tc_partial_body,
      grid=(NB,),
      in_specs=[
          pl.BlockSpec((1, TM, 9), lambda i: (i, 0, 0)),
          pl.BlockSpec((TM, DENSE_W), lambda i: (i, 0)),
          pl.BlockSpec((DENSE_W, H), lambda i: (0, 0)),
          pl.BlockSpec((SMALL, H), lambda i: (0, 0)),
          pl.BlockSpec((TM, H), lambda i: (0, 0)),
      ],
      out_specs=pl.BlockSpec((TM, H), lambda i: (i, 0)),
      out_shape=jax.ShapeDtypeStruct((N, H), jnp.float32),
  )(sidx, dense, w2, p, bp)


def _tc_final_body(*refs):
  cat_refs = refs[:NF]
  part_ref, w1_ref, g_ref, bta_ref, out_ref = refs[NF:]
  x1 = jnp.concatenate([r[...] for r in cat_refs], axis=-1)
  acc = jnp.dot(x1, w1_ref[...], preferred_element_type=jnp.float32)
  acc = acc + part_ref[...]
  m = jnp.mean(acc, axis=-1, keepdims=True)
  d = acc - m
  v = jnp.mean(d * d, axis=-1, keepdims=True)
  out_ref[...] = d * lax.rsqrt(v + 1e-12) * g_ref[...] + bta_ref[...]


def _tc_final(cat_list, part, w1, g, bta):
  return pl.pallas_call(
      _tc_final_body,
      grid=(NB,),
      in_specs=[
          pl.BlockSpec((TM, E), lambda i: (i, 0)) for _ in range(NF)
      ] + [
          pl.BlockSpec((TM, H), lambda i: (i, 0)),
          pl.BlockSpec((CAT_W, H), lambda i: (0, 0)),
          pl.BlockSpec((1, H), lambda i: (0, 0)),
          pl.BlockSpec((1, H), lambda i: (0, 0)),
      ],
      out_specs=pl.BlockSpec((TM, H), lambda i: (i, 0)),
      out_shape=jax.ShapeDtypeStruct((N, H), jnp.float32),
  )(*cat_list, part, w1, g, bta)


def kernel(input_ids, elapsed_time, event_type, product_action, hashed_url,
           price_bucket, number_of_category_hash, category_hash_first_level,
           category_hash_second_level, category_hash_third_level,
           description_vector, image_vector, hour, weekday, weekend,
           query_vector, id_table, elapsed_table, event_table, action_table,
           url_table, price_table, numcat_table, cat1_table, cat2_table,
           cat3_table, hour_table, weekday_table, weekend_table, pos_table,
           W, b, ln_gamma, ln_beta):
  big_idx = [input_ids, category_hash_second_level,
             category_hash_third_level, hashed_url]
  big_tables = [id_table, cat2_table, cat3_table, url_table]
  idx_flat = [x.reshape(N).astype(jnp.int32) for x in big_idx]
  cat = _sc_gather(idx_flat, big_tables)

  dense = jnp.concatenate(
      [description_vector.reshape(N, 50), image_vector.reshape(N, 50),
       query_vector.reshape(N, 50)], axis=-1)
  w1 = jnp.concatenate([W[0:64], W[256:320], W[320:384], W[676:740]], axis=0)
  w2 = jnp.concatenate([W[384:484], W[932:982]], axis=0)
  small = [(price_bucket, price_table, W[64:128]),
           (number_of_category_hash, numcat_table, W[128:192]),
           (category_hash_first_level, cat1_table, W[192:256]),
           (elapsed_time, elapsed_table, W[484:548]),
           (event_type, event_table, W[548:612]),
           (product_action, action_table, W[612:676]),
           (hour, hour_table, W[740:804]),
           (weekday, weekday_table, W[804:868]),
           (weekend, weekend_table, W[868:932])]
  p = jnp.concatenate([t @ w for _, t, w in small], axis=0)
  offs, sidx = 0, []
  for ix, t, _ in small:
    sidx.append(ix.reshape(N).astype(jnp.int32) + offs)
    offs += t.shape[0]
  sidx = jnp.stack(sidx, axis=-1).reshape(NB, TM, 9)

  bp = jnp.tile(pos_table + b[None, :], (TM // L, 1))

  part = _tc_partial(sidx, dense, w2, p, bp)
  out = _tc_final(list(cat), part, w1, ln_gamma.reshape(1, H),
                  ln_beta.reshape(1, H))
  return out.reshape(B, L, H)


# speedup vs baseline: 1.5643x; 1.5643x over previous
import functools

import jax
import jax.numpy as jnp
from jax import lax
from jax.experimental import pallas as pl
from jax.experimental.pallas import tpu as pltpu
from jax.experimental.pallas import tpu_sc as plsc

B, L, E, H = 1024, 50, 64, 256
N = B * L
NF = 4
CAT_W = NF * E
SMALL = 145
NC, NS = 2, 16
NW = NC * NS
BPW = N // NW
CH = 80
NCH = BPW // CH
TM = 800
NB = N // TM
RB = B // NB

S1 = (12, 10, 50, 20)
S2 = (10, 10, 24, 7, 2)


def _sc_gather_body(*refs):
  idxs = refs[:NF]
  tables = refs[NF:2 * NF]
  out_hbm = refs[2 * NF]
  idx_v, rows_v, sem = refs[2 * NF + 1:]
  wid = lax.axis_index("s") * NC + lax.axis_index("c")
  base = pl.multiple_of(wid * BPW, 8)
  for f in range(NF):
    tab = tables[f]
    pltpu.sync_copy(idxs[f].at[pl.ds(base, BPW)], idx_v)

    def _fire(c, carry, tab=tab):
      off = pl.multiple_of(c * CH, 8)
      pltpu.make_async_copy(
          tab.at[idx_v.at[pl.ds(off, CH)]],
          rows_v.at[pl.ds(off, CH), :],
          sem).start()
      return carry

    def _drain(c, carry, tab=tab):
      off = pl.multiple_of(c * CH, 8)
      pltpu.make_async_copy(
          tab.at[idx_v.at[pl.ds(off, CH)]],
          rows_v.at[pl.ds(off, CH), :],
          sem).wait()
      return carry

    lax.fori_loop(0, NCH, _fire, 0, unroll=False)
    lax.fori_loop(0, NCH, _drain, 0, unroll=False)
    pltpu.sync_copy(rows_v, out_hbm.at[pl.ds(base, BPW), pl.ds(f * E, E)])


def _sc_gather(idx_list, tables):
  mesh = plsc.VectorSubcoreMesh(core_axis_name="c", subcore_axis_name="s")
  fn = pl.kernel(
      _sc_gather_body,
      mesh=mesh,
      out_type=jax.ShapeDtypeStruct((N, CAT_W), jnp.float32),
      scratch_types=[
          pltpu.VMEM((BPW,), jnp.int32),
          pltpu.VMEM((BPW, E), jnp.float32),
          pltpu.SemaphoreType.DMA,
      ],
      compiler_params=pltpu.CompilerParams(use_tc_tiling_on_sc=False),
  )
  return fn(*idx_list, *tables)


def _tc_body(cat_ref, d1_ref, d2_ref, d3_ref, s_ref, w1_ref, w2a_ref,
             w2b_ref, w2c_ref, p_ref, bp_ref, g_ref, bta_ref, out_ref):
  acc = jnp.dot(cat_ref[...], w1_ref[...],
                preferred_element_type=jnp.float32)
  acc = acc + jnp.dot(d1_ref[...].reshape(TM, 50), w2a_ref[...],
                      preferred_element_type=jnp.float32)
  acc = acc + jnp.dot(d2_ref[...].reshape(TM, 50), w2b_ref[...],
                      preferred_element_type=jnp.float32)
  acc = acc + jnp.dot(d3_ref[...].reshape(TM, 50), w2c_ref[...],
                      preferred_element_type=jnp.float32)
  s = s_ref[...]
  g1 = s[0, 0:1, :]
  g2 = s[0, 1:2, :]
  iota = lax.broadcasted_iota(jnp.int32, (SMALL, 1), 0)
  oht = jnp.zeros((SMALL, TM), dtype=jnp.float32)
  off = 0
  div = 1
  for sz in S1:
    idx = (g1 // div) % sz + off
    oht = oht + (idx == iota).astype(jnp.float32)
    off += sz
    div *= sz
  div = 1
  for sz in S2:
    idx = (g2 // div) % sz + off
    oht = oht + (idx == iota).astype(jnp.float32)
    off += sz
    div *= sz
  acc = acc + lax.dot_general(oht, p_ref[...], (((0,), (0,)), ((), ())),
                              preferred_element_type=jnp.float32)
  acc = acc + bp_ref[...]
  m = jnp.mean(acc, axis=-1, keepdims=True)
  d = acc - m
  v = jnp.mean(d * d, axis=-1, keepdims=True)
  res = d * lax.rsqrt(v + 1e-12) * g_ref[...] + bta_ref[...]
  out_ref[...] = res.reshape(RB, L, H)


def _tc_fused(cat, d1, d2, d3, sidx, w1, w2a, w2b, w2c, p, bp, g, bta):
  return pl.pallas_call(
      _tc_body,
      grid=(NB,),
      in_specs=[
          pl.BlockSpec((TM, CAT_W), lambda i: (i, 0)),
          pl.BlockSpec((RB, L, 50), lambda i: (i, 0, 0)),
          pl.BlockSpec((RB, L, 50), lambda i: (i, 0, 0)),
          pl.BlockSpec((RB, L, 50), lambda i: (i, 0, 0)),
          pl.BlockSpec((1, 2, TM), lambda i: (i, 0, 0)),
          pl.BlockSpec((CAT_W, H), lambda i: (0, 0)),
          pl.BlockSpec((50, H), lambda i: (0, 0)),
          pl.BlockSpec((50, H), lambda i: (0, 0)),
          pl.BlockSpec((50, H), lambda i: (0, 0)),
          pl.BlockSpec((SMALL, H), lambda i: (0, 0)),
          pl.BlockSpec((TM, H), lambda i: (0, 0)),
          pl.BlockSpec((1, H), lambda i: (0, 0)),
          pl.BlockSpec((1, H), lambda i: (0, 0)),
      ],
      out_specs=pl.BlockSpec((RB, L, H), lambda i: (i, 0, 0)),
      out_shape=jax.ShapeDtypeStruct((B, L, H), jnp.float32),
  )(cat, d1, d2, d3, sidx, w1, w2a, w2b, w2c, p, bp, g, bta)


def kernel(input_ids, elapsed_time, event_type, product_action, hashed_url,
           price_bucket, number_of_category_hash, category_hash_first_level,
           category_hash_second_level, category_hash_third_level,
           description_vector, image_vector, hour, weekday, weekend,
           query_vector, id_table, elapsed_table, event_table, action_table,
           url_table, price_table, numcat_table, cat1_table, cat2_table,
           cat3_table, hour_table, weekday_table, weekend_table, pos_table,
           W, b, ln_gamma, ln_beta):
  big_idx = [input_ids, category_hash_second_level,
             category_hash_third_level, hashed_url]
  big_tables = [id_table, cat2_table, cat3_table, url_table]
  idx_flat = [x.reshape(N).astype(jnp.int32) for x in big_idx]
  cat = _sc_gather(idx_flat, big_tables)

  w1 = jnp.concatenate([W[0:64], W[256:320], W[320:384], W[676:740]], axis=0)
  small = [(price_table, W[64:128]), (numcat_table, W[128:192]),
           (cat1_table, W[192:256]), (elapsed_table, W[484:548]),
           (event_table, W[548:612]), (action_table, W[612:676]),
           (hour_table, W[740:804]), (weekday_table, W[804:868]),
           (weekend_table, W[868:932])]
  p = jnp.concatenate([t @ w for t, w in small], axis=0)
  i32 = lambda x: x.astype(jnp.int32)
  g1 = (i32(price_bucket) + 12 * (i32(number_of_category_hash)
        + 10 * (i32(category_hash_first_level) + 50 * i32(elapsed_time))))
  g2 = (i32(event_type) + 10 * (i32(product_action)
        + 10 * (i32(hour) + 24 * (i32(weekday) + 7 * i32(weekend)))))
  sidx = jnp.stack([g1.reshape(NB, TM), g2.reshape(NB, TM)], axis=1)

  bp = jnp.tile(pos_table + b[None, :], (TM // L, 1))

  return _tc_fused(cat, description_vector, image_vector, query_vector,
                   sidx, w1, W[384:434], W[434:484], W[932:982], p, bp,
                   ln_gamma.reshape(1, H), ln_beta.reshape(1, H))
